# Initial kernel scaffold; baseline (speedup 1.0000x reference)
import jax
import jax.numpy as jnp
from jax.experimental import pallas as pl


def kernel(token_ids, shard_weights):
    raise NotImplementedError("write your pallas kernel here")


# speedup vs baseline: 19.3359x; 19.3359x over previous
import functools

import jax
import jax.numpy as jnp
from jax import lax
from jax.experimental import pallas as pl
from jax.experimental.pallas import tpu as pltpu
from jax.experimental.pallas import tpu_sc as plsc

_SHARD_SIZE = 2048
_NUM_SHARDS = 49
_EMBED_DIM = 64
_BATCH = 4096
_SEQ_LEN = 200

_TOT = _BATCH * _SEQ_LEN
_NW = 32
_PER_W = _TOT // _NW
_CHUNK = 128
_NCH = _PER_W // _CHUNK

_mesh = plsc.VectorSubcoreMesh(core_axis_name="c", subcore_axis_name="s")


@functools.partial(
    pl.kernel,
    out_type=jax.ShapeDtypeStruct((_TOT, _EMBED_DIM), jnp.float32),
    mesh=_mesh,
    compiler_params=pltpu.CompilerParams(use_tc_tiling_on_sc=False),
    scratch_types=[
        pltpu.VMEM((_NCH, _CHUNK), jnp.int32),
        pltpu.VMEM((_CHUNK, _EMBED_DIM), jnp.float32),
        pltpu.VMEM((_CHUNK, _EMBED_DIM), jnp.float32),
        pltpu.SemaphoreType.DMA,
        pltpu.SemaphoreType.DMA,
    ],
)
def _gather_kernel(idx_hbm, table_hbm, out_hbm, idx_v, rows0, rows1, sem0, sem1):
    wid = lax.axis_index("s") * 2 + lax.axis_index("c")
    row_base = wid * _NCH
    out_base = wid * _PER_W

    pltpu.sync_copy(idx_hbm.at[pl.ds(row_base, _NCH)], idx_v)

    bufs = ((rows0, sem0), (rows1, sem1))

    def start_gather(j, b):
        rows, sem = bufs[b]
        pltpu.async_copy(table_hbm.at[idx_v.at[j]], rows, sem)

    def finish(j, b):
        rows, sem = bufs[b]
        pltpu.make_async_copy(table_hbm.at[idx_v.at[j]], rows, sem).wait()
        pltpu.sync_copy(
            rows, out_hbm.at[pl.ds(out_base + j * _CHUNK, _CHUNK)]
        )

    start_gather(0, 0)

    def body(k, _):
        j = 2 * k
        start_gather(j + 1, 1)
        finish(j, 0)

        @pl.when(j + 2 < _NCH)
        def _():
            start_gather(j + 2, 0)

        finish(j + 1, 1)
        return ()

    lax.fori_loop(0, _NCH // 2, body, ())


def kernel(token_ids, shard_weights):
    table = shard_weights.reshape(_NUM_SHARDS * _SHARD_SIZE, _EMBED_DIM)
    idx = token_ids.reshape(_TOT // _CHUNK, _CHUNK)
    out = _gather_kernel(idx, table)
    return out.reshape(_BATCH, _SEQ_LEN, _EMBED_DIM)


# speedup vs baseline: 20.0316x; 1.0360x over previous
import functools

import jax
import jax.numpy as jnp
from jax import lax
from jax.experimental import pallas as pl
from jax.experimental.pallas import tpu as pltpu
from jax.experimental.pallas import tpu_sc as plsc

_SHARD_SIZE = 2048
_NUM_SHARDS = 49
_EMBED_DIM = 64
_BATCH = 4096
_SEQ_LEN = 200

_TOT = _BATCH * _SEQ_LEN
_NW = 32
_PER_W = _TOT // _NW
_CHUNK = 128
_NCH = _PER_W // _CHUNK

_mesh = plsc.VectorSubcoreMesh(core_axis_name="c", subcore_axis_name="s")


_NBUF = 8
_G = 4
_NOUT = _NCH // _NBUF


@functools.partial(
    pl.kernel,
    out_type=jax.ShapeDtypeStruct((_TOT, _EMBED_DIM), jnp.float32),
    mesh=_mesh,
    compiler_params=pltpu.CompilerParams(use_tc_tiling_on_sc=False),
    scratch_types=[
        pltpu.VMEM((_NCH, _CHUNK), jnp.int32),
        pltpu.VMEM((_NBUF, _CHUNK, _EMBED_DIM), jnp.float32),
        [pltpu.SemaphoreType.DMA] * _NBUF,
        [pltpu.SemaphoreType.DMA] * _NBUF,
    ],
)
def _gather_kernel(idx_hbm, table_hbm, out_hbm, idx_v, rows, gsems, ssems):
    wid = lax.axis_index("s") * 2 + lax.axis_index("c")
    row_base = wid * _NCH
    out_base = wid * _PER_W

    pltpu.sync_copy(idx_hbm.at[pl.ds(row_base, _NCH)], idx_v)

    def start_gather(j, b):
        pltpu.async_copy(table_hbm.at[idx_v.at[j]], rows.at[b], gsems[b])

    def wait_gather(j, b):
        pltpu.make_async_copy(
            table_hbm.at[idx_v.at[j]], rows.at[b], gsems[b]
        ).wait()

    def start_store(j, b):
        pltpu.async_copy(
            rows.at[b], out_hbm.at[pl.ds(out_base + j * _CHUNK, _CHUNK)],
            ssems[b],
        )

    def wait_store(j, b):
        pltpu.make_async_copy(
            rows.at[b], out_hbm.at[pl.ds(out_base + j * _CHUNK, _CHUNK)],
            ssems[b],
        ).wait()

    for j in range(_G):
        start_gather(j, j)

    for b in range(_NBUF):
        wait_gather(b, b)
        start_store(b, b)
        bg = (b + _G) % _NBUF
        if b >= _G:
            wait_store(b - _G, bg)
        start_gather(b + _G, bg)

    def body(k, _):
        i0 = k * _NBUF
        for b in range(_NBUF):
            i = i0 + b
            wait_gather(i, b)
            start_store(i, b)
            bg = (b + _G) % _NBUF
            wait_store(i - _G, bg)
            start_gather(i + _G, bg)
        return ()

    lax.fori_loop(1, _NOUT - 1, body, ())

    i0 = (_NOUT - 1) * _NBUF
    for b in range(_NBUF):
        i = i0 + b
        wait_gather(i, b)
        start_store(i, b)
        bg = (b + _G) % _NBUF
        if b < _NBUF - _G:
            wait_store(i - _G, bg)
            start_gather(i + _G, bg)

    for b in range(_NBUF):
        wait_store(i0 + b, b)


def kernel(token_ids, shard_weights):
    table = shard_weights.reshape(_NUM_SHARDS * _SHARD_SIZE, _EMBED_DIM)
    idx = token_ids.reshape(_TOT // _CHUNK, _CHUNK)
    out = _gather_kernel(idx, table)
    return out.reshape(_BATCH, _SEQ_LEN, _EMBED_DIM)
